# baseline (speedup 1.0000x reference)
import functools

import jax
import jax.numpy as jnp
from jax import lax
from jax.experimental import pallas as pl
from jax.experimental.pallas import tpu as pltpu
from jax.experimental.pallas import tpu_sc as plsc


@functools.lru_cache(maxsize=None)
def _build(V, D, B):
    info = plsc.get_sparse_core_info()
    NC, NS, L = info.num_cores, info.num_subcores, info.num_lanes
    NW = NC * NS
    assert B % NW == 0 and D % L == 0
    b_per_w = B // NW
    CHUNK = min(128, b_per_w)
    n_chunks = b_per_w // CHUNK
    assert b_per_w % CHUNK == 0

    mesh = plsc.VectorSubcoreMesh(core_axis_name="c", subcore_axis_name="s")

    @functools.partial(
        pl.kernel,
        mesh=mesh,
        compiler_params=pltpu.CompilerParams(use_tc_tiling_on_sc=False),
        out_type=jax.ShapeDtypeStruct((NW, L), jnp.float32),
        scratch_types=[
            pltpu.VMEM((n_chunks, CHUNK), jnp.int32),
            pltpu.VMEM((b_per_w, D), jnp.float32),
            pltpu.VMEM((b_per_w, D), jnp.float32),
            pltpu.VMEM((L,), jnp.float32),
            pltpu.SemaphoreType.DMA,
        ],
    )
    def k(inputs_hbm, labels_hbm, center_hbm, out_hbm,
          idx_v, rows_v, in_v, acc_v, sem):
        wid = lax.axis_index("s") * NC + lax.axis_index("c")
        base = wid * b_per_w
        pltpu.sync_copy(labels_hbm.at[wid], idx_v)
        copies = []
        for j in range(n_chunks):
            copies.append(pltpu.async_copy(
                center_hbm.at[idx_v.at[j]],
                rows_v.at[pl.ds(j * CHUNK, CHUNK)],
                sem,
            ))
        pltpu.sync_copy(inputs_hbm.at[pl.ds(base, b_per_w)], in_v)
        for cp in copies:
            cp.wait()

        def body(r, acc):
            for c in range(D // L):
                d = in_v[r, pl.ds(c * L, L)] - rows_v[r, pl.ds(c * L, L)]
                acc = acc + d * d
            return acc

        acc = lax.fori_loop(0, b_per_w, body, jnp.zeros((L,), jnp.float32))
        acc_v[...] = acc
        pltpu.sync_copy(acc_v, out_hbm.at[wid])

    return k, NW, n_chunks, CHUNK


def kernel(inputs, labels, center):
    B, D = inputs.shape
    V = center.shape[0]
    del V
    k, NW, n_chunks, CHUNK = _build(center.shape[0], D, B)
    labels3 = labels.astype(jnp.int32).reshape(NW, n_chunks, CHUNK)
    partials = k(inputs, labels3, center)
    return jnp.sum(partials) / jnp.float32(B * D)
